# Initial kernel scaffold; baseline (speedup 1.0000x reference)
import jax
import jax.numpy as jnp
from jax.experimental import pallas as pl


def kernel(batch_rgn_sqn, encoding):
    raise NotImplementedError("write your pallas kernel here")


# speedup vs baseline: 1.0679x; 1.0679x over previous
import functools

import jax
import jax.numpy as jnp
from jax import lax
from jax.experimental import pallas as pl
from jax.experimental.pallas import tpu as pltpu
from jax.experimental.pallas import tpu_sc as plsc

_PS_DIM = 64
_TABLE_ROWS = 16
_NC = 2
_NS = 16
_NW = _NC * _NS
_IPW = 128
_K = 5
_CH = _K * _IPW


def _sc_gather(table, idx3, rows_per_w, nsc):
    mesh = plsc.VectorSubcoreMesh(core_axis_name="c", subcore_axis_name="s")
    n_rows = _NW * rows_per_w

    @functools.partial(
        pl.kernel,
        out_type=jax.ShapeDtypeStruct((n_rows, _PS_DIM), jnp.float32),
        mesh=mesh,
        scratch_types=[
            pltpu.VMEM((rows_per_w // _IPW, _IPW), jnp.int32),
            pltpu.VMEM((_CH, _PS_DIM), jnp.float32),
            pltpu.SemaphoreType.DMA,
        ],
        compiler_params=pltpu.CompilerParams(use_tc_tiling_on_sc=False),
    )
    def k(table_hbm, idx_hbm, out_hbm, idx_v, rows_v, gsem):
        wid = lax.axis_index("s") * _NC + lax.axis_index("c")
        pltpu.sync_copy(idx_hbm.at[wid], idx_v)
        base = wid * rows_per_w

        def body(g, carry):
            descs = []
            for i in range(_K):
                descs.append(
                    pltpu.async_copy(
                        table_hbm.at[idx_v.at[g * _K + i]],
                        rows_v.at[pl.ds(i * _IPW, _IPW)],
                        gsem,
                    )
                )
            for d in descs:
                d.wait()
            pltpu.sync_copy(rows_v, out_hbm.at[pl.ds(base + g * _CH, _CH)])
            return carry

        lax.fori_loop(0, nsc, body, 0)

    return k(table, idx3)


def kernel(batch_rgn_sqn, encoding):
    b, l = batch_rgn_sqn.shape
    n = b * l
    rows_per_w = n // _NW
    assert rows_per_w % _CH == 0
    table = encoding[:_TABLE_ROWS, :_PS_DIM]
    idx3 = batch_rgn_sqn.astype(jnp.int32).reshape(_NW, rows_per_w // _IPW, _IPW)
    out = _sc_gather(table, idx3, rows_per_w, rows_per_w // _CH)
    return out.reshape(b, l, _PS_DIM)


# speedup vs baseline: 1.2618x; 1.1816x over previous
import functools

import jax
import jax.numpy as jnp
from jax import lax
from jax.experimental import pallas as pl
from jax.experimental.pallas import tpu as pltpu
from jax.experimental.pallas import tpu_sc as plsc

_PS_DIM = 64
_TABLE_ROWS = 16
_NC = 2
_NS = 16
_NW = _NC * _NS
_L = 16
_GPC = 40
_CH = _GPC * _L


def _sc_lookup(table, idx3, rows_per_w):
    mesh = plsc.VectorSubcoreMesh(core_axis_name="c", subcore_axis_name="s")
    n_rows = _NW * rows_per_w
    groups_per_w = rows_per_w // _L
    n_chunks = rows_per_w // _CH

    @functools.partial(
        pl.kernel,
        out_type=jax.ShapeDtypeStruct((n_rows * _PS_DIM,), jnp.float32),
        mesh=mesh,
        scratch_types=[
            pltpu.VMEM((_TABLE_ROWS * _PS_DIM,), jnp.float32),
            pltpu.VMEM((groups_per_w, _L), jnp.int32),
            pltpu.VMEM((_CH * _PS_DIM,), jnp.float32),
            pltpu.VMEM((_CH * _PS_DIM,), jnp.float32),
            pltpu.SemaphoreType.DMA,
            pltpu.SemaphoreType.DMA,
        ],
        compiler_params=pltpu.CompilerParams(
            use_tc_tiling_on_sc=False, needs_layout_passes=False
        ),
    )
    def k(table_hbm, idx_hbm, out_hbm, table_v, idx_v, buf0, buf1, sem0, sem1):
        wid = lax.axis_index("s") * _NC + lax.axis_index("c")
        pltpu.sync_copy(table_hbm, table_v)
        pltpu.sync_copy(idx_hbm.at[wid], idx_v)
        base = wid * rows_per_w
        lane_off = lax.iota(jnp.int32, _L) * _PS_DIM
        bufs = (buf0, buf1)
        sems = (sem0, sem1)

        def fill(buf, chunk):
            def group(g, carry):
                src0 = idx_v[chunk * _GPC + g] * _PS_DIM
                dst0 = lane_off + g * (_L * _PS_DIM)
                for c in range(_PS_DIM):
                    v = plsc.load_gather(table_v, [src0 + c])
                    plsc.store_scatter(buf, [dst0 + c], v)
                return carry

            lax.fori_loop(0, _GPC, group, 0)

        def flush(buf, sem, chunk):
            pltpu.async_copy(
                buf,
                out_hbm.at[pl.ds((base + chunk * _CH) * _PS_DIM, _CH * _PS_DIM)],
                sem,
            )

        def drain(buf, sem):
            pltpu.make_async_copy(
                out_hbm.at[pl.ds(base * _PS_DIM, _CH * _PS_DIM)], buf, sem
            ).wait()

        for b in range(2):
            fill(bufs[b], b)
            flush(bufs[b], sems[b], b)

        def outer(g2, carry):
            for b in range(2):
                chunk = g2 * 2 + b
                drain(bufs[b], sems[b])
                fill(bufs[b], chunk)
                flush(bufs[b], sems[b], chunk)
            return carry

        lax.fori_loop(1, n_chunks // 2, outer, 0)
        drain(buf0, sem0)
        drain(buf1, sem1)

    return k(table, idx3)


def kernel(batch_rgn_sqn, encoding):
    b, l = batch_rgn_sqn.shape
    n = b * l
    rows_per_w = n // _NW
    assert rows_per_w % _CH == 0
    table = encoding[:_TABLE_ROWS, :_PS_DIM].reshape(-1)
    idx3 = batch_rgn_sqn.astype(jnp.int32).reshape(_NW, rows_per_w // _L, _L)
    out = _sc_lookup(table, idx3, rows_per_w)
    return out.reshape(b, l, _PS_DIM)


# speedup vs baseline: 1.4551x; 1.1532x over previous
import functools

import jax
import jax.numpy as jnp
from jax import lax
from jax.experimental import pallas as pl
from jax.experimental.pallas import tpu as pltpu
from jax.experimental.pallas import tpu_sc as plsc

_PS_DIM = 64
_TABLE_ROWS = 16
_NC = 2
_NS = 16
_NW = _NC * _NS
_L = 16
_GPC = 40
_CH = _GPC * _L


def _sc_lookup(table, idx3, rows_per_w):
    mesh = plsc.VectorSubcoreMesh(core_axis_name="c", subcore_axis_name="s")
    n_rows = _NW * rows_per_w
    groups_per_w = rows_per_w // _L
    n_chunks = rows_per_w // _CH

    @functools.partial(
        pl.kernel,
        out_type=jax.ShapeDtypeStruct((n_rows * _PS_DIM,), jnp.float32),
        mesh=mesh,
        scratch_types=[
            pltpu.VMEM((_TABLE_ROWS * _PS_DIM,), jnp.float32),
            pltpu.VMEM((groups_per_w, _L), jnp.int32),
            pltpu.VMEM((_CH * _PS_DIM,), jnp.float32),
            pltpu.VMEM((_CH * _PS_DIM,), jnp.float32),
            pltpu.SemaphoreType.DMA,
            pltpu.SemaphoreType.DMA,
        ],
        compiler_params=pltpu.CompilerParams(
            use_tc_tiling_on_sc=False, needs_layout_passes=False
        ),
    )
    def k(table_hbm, idx_hbm, out_hbm, table_v, idx_v, buf0, buf1, sem0, sem1):
        wid = lax.axis_index("s") * _NC + lax.axis_index("c")
        pltpu.sync_copy(table_hbm, table_v)
        pltpu.sync_copy(idx_hbm.at[wid], idx_v)
        base = wid * rows_per_w
        lane_off = lax.iota(jnp.int32, _L) * _PS_DIM
        bufs = (buf0, buf1)
        sems = (sem0, sem1)

        def fill(buf, chunk):
            @plsc.parallel_loop(0, _GPC, unroll=2)
            def group(g):
                src0 = idx_v[chunk * _GPC + g] * _PS_DIM
                dst0 = lane_off + g * (_L * _PS_DIM)
                for c in range(_PS_DIM):
                    v = plsc.load_gather(table_v, [src0 + c])
                    plsc.store_scatter(buf, [dst0 + c], v)

        def flush(buf, sem, chunk):
            pltpu.async_copy(
                buf,
                out_hbm.at[pl.ds((base + chunk * _CH) * _PS_DIM, _CH * _PS_DIM)],
                sem,
            )

        def drain(buf, sem):
            pltpu.make_async_copy(
                out_hbm.at[pl.ds(base * _PS_DIM, _CH * _PS_DIM)], buf, sem
            ).wait()

        for b in range(2):
            fill(bufs[b], b)
            flush(bufs[b], sems[b], b)

        def outer(g2, carry):
            for b in range(2):
                chunk = g2 * 2 + b
                drain(bufs[b], sems[b])
                fill(bufs[b], chunk)
                flush(bufs[b], sems[b], chunk)
            return carry

        lax.fori_loop(1, n_chunks // 2, outer, 0)
        drain(buf0, sem0)
        drain(buf1, sem1)

    return k(table, idx3)


def kernel(batch_rgn_sqn, encoding):
    b, l = batch_rgn_sqn.shape
    n = b * l
    rows_per_w = n // _NW
    assert rows_per_w % _CH == 0
    table = encoding[:_TABLE_ROWS, :_PS_DIM].reshape(-1)
    idx3 = batch_rgn_sqn.astype(jnp.int32).reshape(_NW, rows_per_w // _L, _L)
    out = _sc_lookup(table, idx3, rows_per_w)
    return out.reshape(b, l, _PS_DIM)


# speedup vs baseline: 2.3458x; 1.6121x over previous
import functools

import jax
import jax.numpy as jnp
from jax import lax
from jax.experimental import pallas as pl
from jax.experimental.pallas import tpu as pltpu
from jax.experimental.pallas import tpu_sc as plsc

_PS_DIM = 64
_TABLE_ROWS = 16
_NC = 2
_NS = 16
_NW = _NC * _NS
_L = 16
_GPC = 40
_CH = _GPC * _L


def _sc_lookup(table, idx3, rows_per_w):
    mesh = plsc.VectorSubcoreMesh(core_axis_name="c", subcore_axis_name="s")
    n_rows = _NW * rows_per_w
    groups_per_w = rows_per_w // _L
    n_chunks = rows_per_w // _CH

    @functools.partial(
        pl.kernel,
        out_type=jax.ShapeDtypeStruct((n_rows, _PS_DIM), jnp.float32),
        mesh=mesh,
        scratch_types=[
            pltpu.VMEM((_TABLE_ROWS * _PS_DIM,), jnp.float32),
            pltpu.VMEM((groups_per_w, _L), jnp.int32),
            pltpu.VMEM((_CH, _PS_DIM), jnp.float32),
            pltpu.VMEM((_CH, _PS_DIM), jnp.float32),
            pltpu.SemaphoreType.DMA,
            pltpu.SemaphoreType.DMA,
        ],
        compiler_params=pltpu.CompilerParams(
            use_tc_tiling_on_sc=False, needs_layout_passes=False
        ),
    )
    def k(table_hbm, idx_hbm, out_hbm, table_v, idx_v, buf0, buf1, sem0, sem1):
        wid = lax.axis_index("s") * _NC + lax.axis_index("c")
        pltpu.sync_copy(table_hbm, table_v)
        pltpu.sync_copy(idx_hbm.at[wid], idx_v)
        base = wid * rows_per_w
        lane_iota = lax.iota(jnp.int32, _L)
        bufs = (buf0, buf1)
        sems = (sem0, sem1)

        def fill(buf, chunk):
            def group(g, carry):
                src0 = idx_v[chunk * _GPC + g] * _PS_DIM
                rows = lane_iota + g * _L

                @plsc.parallel_loop(0, _PS_DIM, unroll=8)
                def col(c):
                    v = plsc.load_gather(table_v, [src0 + c])
                    plsc.store_scatter(
                        buf, [rows, jnp.full((_L,), 0, jnp.int32) + c], v
                    )

                return carry

            lax.fori_loop(0, _GPC, group, 0)

        def flush(buf, sem, chunk):
            pltpu.async_copy(
                buf, out_hbm.at[pl.ds(base + chunk * _CH, _CH)], sem
            )

        def drain(buf, sem):
            pltpu.make_async_copy(out_hbm.at[pl.ds(base, _CH)], buf, sem).wait()

        for b in range(2):
            fill(bufs[b], b)
            flush(bufs[b], sems[b], b)

        def outer(g2, carry):
            for b in range(2):
                chunk = g2 * 2 + b
                drain(bufs[b], sems[b])
                fill(bufs[b], chunk)
                flush(bufs[b], sems[b], chunk)
            return carry

        lax.fori_loop(1, n_chunks // 2, outer, 0)
        drain(buf0, sem0)
        drain(buf1, sem1)

    return k(table, idx3)


def kernel(batch_rgn_sqn, encoding):
    b, l = batch_rgn_sqn.shape
    n = b * l
    rows_per_w = n // _NW
    assert rows_per_w % _CH == 0
    table = encoding[:_TABLE_ROWS, :_PS_DIM].reshape(-1)
    idx3 = batch_rgn_sqn.astype(jnp.int32).reshape(_NW, rows_per_w // _L, _L)
    out = _sc_lookup(table, idx3, rows_per_w)
    return out.reshape(b, l, _PS_DIM)


# speedup vs baseline: 5.0357x; 2.1467x over previous
import functools

import jax
import jax.numpy as jnp
from jax import lax
from jax.experimental import pallas as pl
from jax.experimental.pallas import tpu as pltpu
from jax.experimental.pallas import tpu_sc as plsc

_PS_DIM = 64
_TABLE_ROWS = 16
_NC = 2
_NS = 16
_NW = _NC * _NS
_IPW = 128
_K = 5
_CH = _K * _IPW


def _sc_lookup(table, idx3, rows_per_w):
    mesh = plsc.VectorSubcoreMesh(core_axis_name="c", subcore_axis_name="s")
    n_rows = _NW * rows_per_w
    n_chunks = rows_per_w // _CH

    @functools.partial(
        pl.kernel,
        out_type=jax.ShapeDtypeStruct((n_rows, _PS_DIM), jnp.float32),
        mesh=mesh,
        scratch_types=[
            pltpu.VMEM_SHARED((_TABLE_ROWS, _PS_DIM), jnp.float32),
            pltpu.VMEM((rows_per_w // _IPW, _IPW), jnp.int32),
            pltpu.VMEM((_CH, _PS_DIM), jnp.float32),
            pltpu.VMEM((_CH, _PS_DIM), jnp.float32),
            pltpu.SemaphoreType.DMA,
            pltpu.SemaphoreType.DMA,
            pltpu.SemaphoreType.DMA,
        ],
        compiler_params=pltpu.CompilerParams(
            use_tc_tiling_on_sc=False, needs_layout_passes=False
        ),
    )
    def k(table_hbm, idx_hbm, out_hbm, table_sh, idx_v, buf0, buf1,
          gsem, sem0, sem1):
        sid = lax.axis_index("s")
        wid = sid * _NC + lax.axis_index("c")

        @pl.when(sid == 0)
        def _():
            pltpu.sync_copy(table_hbm, table_sh)

        pltpu.sync_copy(idx_hbm.at[wid], idx_v)
        plsc.subcore_barrier()
        base = wid * rows_per_w
        bufs = (buf0, buf1)
        sems = (sem0, sem1)

        def fill(buf, chunk):
            descs = []
            for i in range(_K):
                descs.append(
                    pltpu.async_copy(
                        table_sh.at[idx_v.at[chunk * _K + i]],
                        buf.at[pl.ds(i * _IPW, _IPW)],
                        gsem,
                    )
                )
            for d in descs:
                d.wait()

        def flush(buf, sem, chunk):
            pltpu.async_copy(
                buf, out_hbm.at[pl.ds(base + chunk * _CH, _CH)], sem
            )

        def drain(buf, sem):
            pltpu.make_async_copy(out_hbm.at[pl.ds(base, _CH)], buf, sem).wait()

        for b in range(2):
            fill(bufs[b], b)
            flush(bufs[b], sems[b], b)

        def outer(g2, carry):
            for b in range(2):
                chunk = g2 * 2 + b
                drain(bufs[b], sems[b])
                fill(bufs[b], chunk)
                flush(bufs[b], sems[b], chunk)
            return carry

        lax.fori_loop(1, n_chunks // 2, outer, 0)
        drain(buf0, sem0)
        drain(buf1, sem1)

    return k(table, idx3)


def kernel(batch_rgn_sqn, encoding):
    b, l = batch_rgn_sqn.shape
    n = b * l
    rows_per_w = n // _NW
    assert rows_per_w % _CH == 0
    table = encoding[:_TABLE_ROWS, :_PS_DIM]
    idx3 = batch_rgn_sqn.astype(jnp.int32).reshape(_NW, rows_per_w // _IPW, _IPW)
    out = _sc_lookup(table, idx3, rows_per_w)
    return out.reshape(b, l, _PS_DIM)
